# Initial kernel scaffold; baseline (speedup 1.0000x reference)
import jax
import jax.numpy as jnp
from jax.experimental import pallas as pl


def kernel(pedxy, hidden_states, cell_states, outputs, grids, node_ids, W_in, b_in, W_t, b_t, W_ih, b_ih, W_hh, b_hh, W_out, b_out):
    raise NotImplementedError("write your pallas kernel here")


# speedup vs baseline: 2.9148x; 2.9148x over previous
import jax
import jax.numpy as jnp
from jax.experimental import pallas as pl
from jax.experimental.pallas import tpu as pltpu

_SEQ = 20
_N = 256
_RNN = 128
_G = 4
_G2 = _G * _G
_EMB = 64
_INP = 2
_OUT = 5


def _body(gt_ref, xy_ref, idr_ref, idc_ref, h0_ref, c0_ref, win_ref, bin_ref,
          wt_ref, bt_ref, wih_ref, bih_ref, whh_ref, bhh_ref, wout_ref,
          bout_ref, out_ref, h_ref, c_ref):
    t = pl.program_id(0)

    @pl.when(t == 0)
    def _init():
        h_ref[...] = h0_ref[...]
        c_ref[...] = c0_ref[...]

    idx_row = idr_ref[0]
    idx_col = idc_ref[0]
    iota0 = jax.lax.broadcasted_iota(jnp.int32, (_N, _N), 0)
    iota1 = jax.lax.broadcasted_iota(jnp.int32, (_N, _N), 1)
    q_mat = (idx_col == iota1).astype(jnp.float32)
    qt_mat = (iota0 == idx_row).astype(jnp.float32)

    hi = jax.lax.Precision.HIGHEST
    h_cur = jnp.dot(q_mat, h_ref[...], precision=hi)
    c_cur = jnp.dot(q_mat, c_ref[...], precision=hi)
    x_cur = jnp.dot(q_mat, xy_ref[0], precision=hi)

    s2 = jnp.dot(gt_ref[0], h_cur)
    social = jnp.concatenate(
        [s2[g * _N:(g + 1) * _N] for g in range(_G2)], axis=1)

    inp_emb = jax.nn.relu(jnp.dot(x_cur, win_ref[...]) + bin_ref[...])
    ten_emb = jax.nn.relu(jnp.dot(social, wt_ref[...]) + bt_ref[...])
    concat = jnp.concatenate([inp_emb, ten_emb], axis=1)

    gates = (jnp.dot(concat, wih_ref[...]) + bih_ref[...]
             + jnp.dot(h_cur, whh_ref[...]) + bhh_ref[...])
    gi = jax.nn.sigmoid(gates[:, :_RNN])
    gf = jax.nn.sigmoid(gates[:, _RNN:2 * _RNN])
    gg = jnp.tanh(gates[:, 2 * _RNN:3 * _RNN])
    go = jax.nn.sigmoid(gates[:, 3 * _RNN:])
    c_new = gf * c_cur + gi * gg
    h_new = go * jnp.tanh(c_new)
    out_t = jnp.dot(h_new, wout_ref[...]) + bout_ref[...]

    out_ref[0] = jnp.dot(qt_mat, out_t, precision=hi)
    h_ref[...] = jnp.dot(qt_mat, h_new, precision=hi)
    c_ref[...] = jnp.dot(qt_mat, c_new, precision=hi)


def kernel(pedxy, hidden_states, cell_states, outputs, grids, node_ids,
           W_in, b_in, W_t, b_t, W_ih, b_ih, W_hh, b_hh, W_out, b_out):
    del outputs

    gt = grids.transpose(0, 3, 1, 2).reshape(_SEQ, _G2 * _N, _N)
    idr = node_ids.reshape(_SEQ, 1, _N)
    idc = node_ids.reshape(_SEQ, _N, 1)

    full = lambda shape: pl.BlockSpec(shape, lambda t: (0,) * len(shape))
    per_t = lambda shape: pl.BlockSpec(shape, lambda t: (t,) + (0,) * (len(shape) - 1))

    outputs_r, h_out, c_out = pl.pallas_call(
        _body,
        grid=(_SEQ,),
        in_specs=[
            per_t((1, _G2 * _N, _N)),
            per_t((1, _N, _INP)),
            per_t((1, 1, _N)),
            per_t((1, _N, 1)),
            full((_N, _RNN)),
            full((_N, _RNN)),
            full((_INP, _EMB)),
            full((1, _EMB)),
            full((_G2 * _RNN, _EMB)),
            full((1, _EMB)),
            full((2 * _EMB, 4 * _RNN)),
            full((1, 4 * _RNN)),
            full((_RNN, 4 * _RNN)),
            full((1, 4 * _RNN)),
            full((_RNN, _OUT)),
            full((1, _OUT)),
        ],
        out_specs=(
            per_t((1, _N, _OUT)),
            full((_N, _RNN)),
            full((_N, _RNN)),
        ),
        out_shape=(
            jax.ShapeDtypeStruct((_SEQ, _N, _OUT), jnp.float32),
            jax.ShapeDtypeStruct((_N, _RNN), jnp.float32),
            jax.ShapeDtypeStruct((_N, _RNN), jnp.float32),
        ),
        compiler_params=pltpu.CompilerParams(
            dimension_semantics=("arbitrary",)),
    )(gt, pedxy, idr, idc, hidden_states,
      cell_states, W_in, b_in.reshape(1, _EMB), W_t, b_t.reshape(1, _EMB),
      W_ih.T, b_ih.reshape(1, 4 * _RNN), W_hh.T, b_hh.reshape(1, 4 * _RNN),
      W_out.T, b_out.reshape(1, _OUT))

    return outputs_r, h_out, c_out
